# baseline (speedup 1.0000x reference)
import functools

import jax
import jax.numpy as jnp
from jax import lax
from jax.experimental import pallas as pl
from jax.experimental.pallas import tpu as pltpu
from jax.experimental.pallas import tpu_sc as plsc

BATCH = 16384
EMB = 16

_NC, _NS = 2, 16
_NW = _NC * _NS
_BPW = BATCH // _NW
_NG = _BPW // 16


def _transpose_rows(rows_v, obuf_v):
    lanes = lax.iota(jnp.int32, 16)
    for g in range(_NG):
        rows = g * 16 + lanes
        for c in range(EMB):
            col = jnp.full((16,), c, jnp.int32)
            val = plsc.load_gather(rows_v, [rows, col])
            obuf_v[c, pl.ds(g * 16, 16)] = val


def _sc_gather(user_hbm, item_hbm, ut_hbm, it_hbm, ue_hbm, ie_hbm,
               uidx_v, iidx_v, urows_v, irows_v, uobuf_v, iobuf_v,
               sem_u, sem_i):
    wid = lax.axis_index("s") * _NC + lax.axis_index("c")
    base = wid * _BPW
    pltpu.sync_copy(user_hbm.at[pl.ds(base, _BPW)], uidx_v)
    pltpu.sync_copy(item_hbm.at[pl.ds(base, _BPW)], iidx_v)
    cu = pltpu.async_copy(ut_hbm.at[uidx_v], urows_v, sem_u)
    ci = pltpu.async_copy(it_hbm.at[iidx_v], irows_v, sem_i)
    cu.wait()
    _transpose_rows(urows_v, uobuf_v)
    pltpu.sync_copy(uobuf_v, ue_hbm.at[:, pl.ds(base, _BPW)])
    ci.wait()
    _transpose_rows(irows_v, iobuf_v)
    pltpu.sync_copy(iobuf_v, ie_hbm.at[:, pl.ds(base, _BPW)])


@functools.cache
def _gather_call():
    return pl.kernel(
        _sc_gather,
        mesh=plsc.VectorSubcoreMesh(core_axis_name="c", subcore_axis_name="s"),
        compiler_params=pltpu.CompilerParams(
            use_tc_tiling_on_sc=False,
            needs_layout_passes=False,
        ),
        out_type=[
            jax.ShapeDtypeStruct((EMB, BATCH), jnp.float32),
            jax.ShapeDtypeStruct((EMB, BATCH), jnp.float32),
        ],
        scratch_types=[
            pltpu.VMEM((_BPW,), jnp.int32),
            pltpu.VMEM((_BPW,), jnp.int32),
            pltpu.VMEM((_BPW, EMB), jnp.float32),
            pltpu.VMEM((_BPW, EMB), jnp.float32),
            pltpu.VMEM((EMB, _BPW), jnp.float32),
            pltpu.VMEM((EMB, _BPW), jnp.float32),
            pltpu.SemaphoreType.DMA,
            pltpu.SemaphoreType.DMA,
        ],
    )


def _mlp_body(ue_ref, ie_ref, w1u_ref, w1i_ref, b1_ref, w2_ref, b2_ref,
              wo_ref, bo_ref, out_ref):
    h = (
        jnp.dot(w1u_ref[...], ue_ref[...], preferred_element_type=jnp.float32)
        + jnp.dot(w1i_ref[...], ie_ref[...], preferred_element_type=jnp.float32)
        + b1_ref[...]
    )
    h = jnp.maximum(h, 0.0)
    h = jnp.dot(w2_ref[...], h, preferred_element_type=jnp.float32) + b2_ref[...]
    h = jnp.maximum(h, 0.0)
    logits = jnp.dot(wo_ref[...], h, preferred_element_type=jnp.float32) + bo_ref[...]
    out_ref[...] = jax.nn.sigmoid(logits)


def kernel(user, item, user_table, item_table, W1, b1, W2, b2, Wo, bo):
    ue_t, ie_t = _gather_call()(user, item, user_table, item_table)
    out = pl.pallas_call(
        _mlp_body,
        out_shape=jax.ShapeDtypeStruct((1, BATCH), jnp.float32),
    )(
        ue_t,
        ie_t,
        W1[:, :EMB],
        W1[:, EMB:],
        b1.reshape(-1, 1),
        W2,
        b2.reshape(-1, 1),
        Wo,
        bo.reshape(1, 1),
    )
    return out.reshape(BATCH)


# speedup vs baseline: 2.3374x; 2.3374x over previous
import functools

import jax
import jax.numpy as jnp
from jax import lax
from jax.experimental import pallas as pl
from jax.experimental.pallas import tpu as pltpu
from jax.experimental.pallas import tpu_sc as plsc

BATCH = 16384
EMB = 16
GRP = 8
PACKED_ROWS = BATCH // GRP

_NC, _NS = 2, 16
_NW = _NC * _NS
_BPW = BATCH // _NW
_CH = 64
_NCH = _BPW // _CH


def _gather_one_table(idx_v, tab3_hbm, big_v, obuf_v, sem):
    def chunk(ci, carry):
        del carry
        base = ci * _CH
        copies = []
        for u in range(_CH // 16):
            idxs = idx_v[pl.ds(base + u * 16, 16)]
            for j in range(16):
                grp = lax.shift_right_logical(idxs[j], 3)
                copies.append(
                    pltpu.make_async_copy(
                        tab3_hbm.at[grp], big_v.at[u * 16 + j], sem))
        for cp in copies:
            cp.start()
        for cp in copies:
            cp.wait()
        for u in range(_CH // 16):
            idxs = idx_v[pl.ds(base + u * 16, 16)]
            for j in range(16):
                r = u * 16 + j
                sub = lax.bitwise_and(idxs[j], 7)
                row = big_v[r, sub, :]
                line = (base + r) // GRP
                obuf_v[line, pl.ds((r % GRP) * EMB, EMB)] = row
        return 0

    lax.fori_loop(0, _NCH, chunk, 0)


def _sc_gather(user_hbm, item_hbm, ut_hbm, it_hbm, ue_hbm, ie_hbm,
               uidx_v, iidx_v, big_v, uobuf_v, iobuf_v, sem):
    wid = lax.axis_index("s") * _NC + lax.axis_index("c")
    base = wid * _BPW
    lines = _BPW // GRP
    pltpu.sync_copy(user_hbm.at[pl.ds(base, _BPW)], uidx_v)
    pltpu.sync_copy(item_hbm.at[pl.ds(base, _BPW)], iidx_v)
    _gather_one_table(uidx_v, ut_hbm, big_v, uobuf_v, sem)
    pltpu.sync_copy(uobuf_v, ue_hbm.at[pl.ds(wid * lines, lines)])
    _gather_one_table(iidx_v, it_hbm, big_v, iobuf_v, sem)
    pltpu.sync_copy(iobuf_v, ie_hbm.at[pl.ds(wid * lines, lines)])


@functools.cache
def _gather_call():
    return pl.kernel(
        _sc_gather,
        mesh=plsc.VectorSubcoreMesh(core_axis_name="c", subcore_axis_name="s"),
        out_type=[
            jax.ShapeDtypeStruct((PACKED_ROWS, GRP * EMB), jnp.float32),
            jax.ShapeDtypeStruct((PACKED_ROWS, GRP * EMB), jnp.float32),
        ],
        scratch_types=[
            pltpu.VMEM((_BPW,), jnp.int32),
            pltpu.VMEM((_BPW,), jnp.int32),
            pltpu.VMEM((_CH, GRP, EMB), jnp.float32),
            pltpu.VMEM((_BPW // GRP, GRP * EMB), jnp.float32),
            pltpu.VMEM((_BPW // GRP, GRP * EMB), jnp.float32),
            pltpu.SemaphoreType.DMA,
        ],
    )


def _mlp_body(ue_ref, ie_ref, w1u_ref, w1i_ref, b1_ref, w2_ref, b2_ref,
              wsel_ref, bo_ref, out_ref):
    h = (
        jnp.dot(ue_ref[...], w1u_ref[...], preferred_element_type=jnp.float32)
        + jnp.dot(ie_ref[...], w1i_ref[...], preferred_element_type=jnp.float32)
        + b1_ref[...]
    )
    h = jnp.maximum(h, 0.0)
    h = jnp.dot(h, w2_ref[...], preferred_element_type=jnp.float32) + b2_ref[...]
    h = jnp.maximum(h, 0.0)
    logits = jnp.dot(h, wsel_ref[...], preferred_element_type=jnp.float32) + bo_ref[...]
    out_ref[...] = jax.nn.sigmoid(logits)


def kernel(user, item, user_table, item_table, W1, b1, W2, b2, Wo, bo):
    ut3 = user_table.reshape(-1, GRP, EMB)
    it3 = item_table.reshape(-1, GRP, EMB)
    ue_p, ie_p = _gather_call()(user, item, ut3, it3)

    eye = jnp.eye(GRP, dtype=jnp.float32)
    w1u = jnp.kron(eye, W1[:, :EMB].T)
    w1i = jnp.kron(eye, W1[:, EMB:].T)
    b1t = jnp.tile(jnp.pad(b1, (0, 0)), (GRP,)).reshape(1, -1)
    w2p = jnp.pad(W2.T, ((0, 0), (0, EMB - W2.shape[0])))
    w2k = jnp.kron(eye, w2p)
    b2t = jnp.tile(jnp.pad(b2, (0, EMB - b2.shape[0])), (GRP,)).reshape(1, -1)
    wsel = jnp.kron(eye, jnp.pad(Wo.T, ((0, EMB - Wo.shape[1]), (0, 0)))).reshape(
        GRP * EMB, GRP)
    bot = jnp.broadcast_to(bo.reshape(1, 1), (1, GRP))

    out = pl.pallas_call(
        _mlp_body,
        out_shape=jax.ShapeDtypeStruct((PACKED_ROWS, GRP), jnp.float32),
    )(ue_p, ie_p, w1u, w1i, b1t, w2k, b2t, wsel, bot)
    return out.reshape(BATCH)
